# baseline (speedup 1.0000x reference)
import functools

import jax
import jax.numpy as jnp
from jax import lax
from jax.experimental import pallas as pl
from jax.experimental.pallas import tpu as pltpu
from jax.experimental.pallas import tpu_sc as plsc

_NC, _NS, _LANES = 2, 16, 16
_NBUF = 4
_BB = 128


@jax.jit
def kernel(x, emb_table, pos_table):
    B, L = x.shape
    V, H = emb_table.shape
    x4 = (x.astype(jnp.int32).T.reshape(L // 8, 8, B // _BB, _BB)
          .transpose(0, 2, 1, 3))
    pos = pos_table[:L].astype(jnp.float32)
    out5 = _build(B, L, H)(x4, emb_table, pos)
    return out5.transpose(2, 4, 0, 1, 3).reshape(B, L, H)


@functools.lru_cache(maxsize=None)
def _build(B, L, H):
    NW = _NC * _NS
    assert B == _BB * NW, (B, NW)
    assert H % _LANES == 0 and _BB % _LANES == 0
    assert L % 8 == 0
    assert L >= 2 * _NBUF and (L - 1 - (_NBUF - 1)) % _NBUF == 0
    HG = H // 8
    mesh = plsc.VectorSubcoreMesh(core_axis_name="c", subcore_axis_name="s")

    scratch = [
        pltpu.VMEM((L, H), jnp.float32),
        pltpu.VMEM((_NBUF, _BB), jnp.int32),
        pltpu.VMEM((_NBUF, _BB, H), jnp.float32),
        pltpu.VMEM((_NBUF, HG, 8, _BB + 1), jnp.float32),
    ] + [pltpu.SemaphoreType.DMA] * (3 * _NBUF)

    @functools.partial(
        pl.kernel,
        out_type=jax.ShapeDtypeStruct((L, HG, B // _BB, 8, _BB),
                                      jnp.float32),
        mesh=mesh,
        scratch_types=scratch,
        compiler_params=pltpu.CompilerParams(use_tc_tiling_on_sc=False,
                                             needs_layout_passes=False,
                                             disable_bounds_checks=True),
    )
    def k(xt_hbm, emb_hbm, pos_hbm, out_hbm, pos_v, idx_v, rows_v, tr_v,
          *sems):
        sem_i = sems[0:_NBUF]
        sem_g = sems[_NBUF:2 * _NBUF]
        sem_s = sems[2 * _NBUF:3 * _NBUF]
        wid = lax.axis_index("s") * _NC + lax.axis_index("c")
        pltpu.sync_copy(pos_hbm, pos_v)

        iot = lax.iota(jnp.int32, _LANES)
        h1c = [(q * _LANES + iot) // 8 for q in range(H // _LANES)]
        h2c = [(q * _LANES + iot) % 8 for q in range(H // _LANES)]

        def fire_idx(l, s):
            pltpu.async_copy(xt_hbm.at[l // 8, wid, l % 8], idx_v.at[s],
                             sem_i[s])

        def wait_idx(s):
            pltpu.make_async_copy(xt_hbm.at[0, 0, 0], idx_v.at[s],
                                  sem_i[s]).wait()

        def fire_gather(s):
            pltpu.async_copy(emb_hbm.at[idx_v.at[s]], rows_v.at[s], sem_g[s])

        def wait_gather(s):
            pltpu.make_async_copy(out_hbm.at[0, :, 0], rows_v.at[s],
                                  sem_g[s]).wait()

        def fire_store(l, s):
            pltpu.async_copy(tr_v.at[s, :, :, pl.ds(0, _BB)],
                             out_hbm.at[l, :, wid], sem_s[s])

        def wait_store(s):
            pltpu.make_async_copy(tr_v.at[s, :, :, pl.ds(0, _BB)],
                                  out_hbm.at[0, :, 0], sem_s[s]).wait()

        def transpose_add(l, s):
            pos_q = [pos_v[l, pl.ds(q * _LANES, _LANES)]
                     for q in range(H // _LANES)]
            tr = tr_v.at[s]

            @plsc.parallel_loop(0, _BB, unroll=2)
            def body_t(t):
                tspl = jnp.broadcast_to(t, (_LANES,))
                for q in range(H // _LANES):
                    v = rows_v[s, t, pl.ds(q * _LANES, _LANES)] + pos_q[q]
                    plsc.store_scatter(tr, [h1c[q], h2c[q], tspl], v)

        for j in range(_NBUF - 1):
            fire_idx(j, j)
            wait_idx(j)
            fire_gather(j)

        def step(l, s, prefetch, first=False):
            sp = (s + _NBUF - 1) % _NBUF
            if prefetch:
                fire_idx(l + (_NBUF - 1), sp)
            wait_gather(s)
            transpose_add(l, s)
            fire_store(l, s)
            if prefetch:
                if not first:
                    wait_store(sp)
                wait_idx(sp)
                fire_gather(sp)

        step(0, 0, prefetch=True, first=True)

        n_main = L - 1 - (_NBUF - 1)

        def main_body(t, c):
            for u in range(_NBUF):
                l = 1 + t * _NBUF + u
                step(l, (1 + u) % _NBUF, prefetch=True)
            return c

        lax.fori_loop(0, n_main // _NBUF, main_body, 0)

        for l in range(L - (_NBUF - 1), L):
            step(l, l % _NBUF, prefetch=False)

        for s in range(_NBUF):
            wait_store(s)

    return k


# speedup vs baseline: 1.0061x; 1.0061x over previous
import functools

import jax
import jax.numpy as jnp
from jax import lax
from jax.experimental import pallas as pl
from jax.experimental.pallas import tpu as pltpu
from jax.experimental.pallas import tpu_sc as plsc

_NC, _NS, _LANES = 2, 16, 16
_NBUF = 5
_BB = 128


@jax.jit
def kernel(x, emb_table, pos_table):
    B, L = x.shape
    V, H = emb_table.shape
    x4 = (x.astype(jnp.int32).T.reshape(L // 8, 8, B // _BB, _BB)
          .transpose(0, 2, 1, 3))
    pos = pos_table[:L].astype(jnp.float32)
    out5 = _build(B, L, H)(x4, emb_table, pos)
    return out5.transpose(2, 4, 0, 1, 3).reshape(B, L, H)


@functools.lru_cache(maxsize=None)
def _build(B, L, H):
    NW = _NC * _NS
    assert B == _BB * NW, (B, NW)
    assert H % _LANES == 0 and _BB % _LANES == 0
    assert L % 8 == 0
    assert L >= 2 * _NBUF and (L - 1 - (_NBUF - 1)) % _NBUF == 0
    HG = H // 8
    mesh = plsc.VectorSubcoreMesh(core_axis_name="c", subcore_axis_name="s")

    scratch = [
        pltpu.VMEM((L, H), jnp.float32),
        pltpu.VMEM((_NBUF, _BB), jnp.int32),
        pltpu.VMEM((_NBUF, _BB, H), jnp.float32),
        pltpu.VMEM((_NBUF, HG, 8, _BB + 1), jnp.float32),
    ] + [pltpu.SemaphoreType.DMA] * (3 * _NBUF)

    @functools.partial(
        pl.kernel,
        out_type=jax.ShapeDtypeStruct((L, HG, B // _BB, 8, _BB),
                                      jnp.float32),
        mesh=mesh,
        scratch_types=scratch,
        compiler_params=pltpu.CompilerParams(use_tc_tiling_on_sc=False,
                                             needs_layout_passes=False,
                                             disable_bounds_checks=True),
    )
    def k(xt_hbm, emb_hbm, pos_hbm, out_hbm, pos_v, idx_v, rows_v, tr_v,
          *sems):
        sem_i = sems[0:_NBUF]
        sem_g = sems[_NBUF:2 * _NBUF]
        sem_s = sems[2 * _NBUF:3 * _NBUF]
        wid = lax.axis_index("s") * _NC + lax.axis_index("c")
        pltpu.sync_copy(pos_hbm, pos_v)

        iot = lax.iota(jnp.int32, _LANES)
        h1c = [(q * _LANES + iot) // 8 for q in range(H // _LANES)]
        h2c = [(q * _LANES + iot) % 8 for q in range(H // _LANES)]

        def fire_idx(l, s):
            pltpu.async_copy(xt_hbm.at[l // 8, wid, l % 8], idx_v.at[s],
                             sem_i[s])

        def wait_idx(s):
            pltpu.make_async_copy(xt_hbm.at[0, 0, 0], idx_v.at[s],
                                  sem_i[s]).wait()

        def fire_gather(s):
            pltpu.async_copy(emb_hbm.at[idx_v.at[s]], rows_v.at[s], sem_g[s])

        def wait_gather(s):
            pltpu.make_async_copy(out_hbm.at[0, :, 0], rows_v.at[s],
                                  sem_g[s]).wait()

        def fire_store(l, s):
            pltpu.async_copy(tr_v.at[s, :, :, pl.ds(0, _BB)],
                             out_hbm.at[l, :, wid], sem_s[s])

        def wait_store(s):
            pltpu.make_async_copy(tr_v.at[s, :, :, pl.ds(0, _BB)],
                                  out_hbm.at[0, :, 0], sem_s[s]).wait()

        def transpose_add(l, s):
            pos_q = [pos_v[l, pl.ds(q * _LANES, _LANES)]
                     for q in range(H // _LANES)]
            tr = tr_v.at[s]

            @plsc.parallel_loop(0, _BB, unroll=4)
            def body_t(t):
                tspl = jnp.broadcast_to(t, (_LANES,))
                for q in range(H // _LANES):
                    v = rows_v[s, t, pl.ds(q * _LANES, _LANES)] + pos_q[q]
                    plsc.store_scatter(tr, [h1c[q], h2c[q], tspl], v)

        for j in range(_NBUF - 1):
            fire_idx(j, j)
            wait_idx(j)
            fire_gather(j)

        def step(l, s, prefetch, first=False):
            sp = (s + _NBUF - 1) % _NBUF
            if prefetch:
                fire_idx(l + (_NBUF - 1), sp)
            wait_gather(s)
            transpose_add(l, s)
            fire_store(l, s)
            if prefetch:
                if not first:
                    wait_store(sp)
                wait_idx(sp)
                fire_gather(sp)

        step(0, 0, prefetch=True, first=True)

        n_main = L - 1 - (_NBUF - 1)

        def main_body(t, c):
            for u in range(_NBUF):
                l = 1 + t * _NBUF + u
                step(l, (1 + u) % _NBUF, prefetch=True)
            return c

        lax.fori_loop(0, n_main // _NBUF, main_body, 0)

        for l in range(L - (_NBUF - 1), L):
            step(l, l % _NBUF, prefetch=False)

        for s in range(_NBUF):
            wait_store(s)

    return k
